# baseline (speedup 1.0000x reference)
import numpy as np
import jax
import jax.numpy as jnp
from jax import lax
from jax.experimental import pallas as pl
from jax.experimental.pallas import tpu as pltpu
from jax.experimental.pallas import tpu_sc as plsc

_N = 65536
_C = 1000
_NBINS = 20
_BIN_SIZE = _N // _NBINS
_BIN_LOWER = tuple(
    int(v) for v in np.linspace(0, _N - _BIN_SIZE, _NBINS).astype(np.int32)
)

_DENSE_ONLY = False
_NT = 49152
_NSC = _N - _NT
_W = 32
_RW = _NSC // _W
_SC_BLK = 16
_ROWS = 4096
_G = _NT // _ROWS
_SR, _SL = 512, 128


def _dense_body(lab_ref, x_ref, key_ref, pay_ref):
    x = x_ref[...]
    m = jnp.max(x, axis=1, keepdims=True)
    s = jnp.sum(jnp.exp(x - m), axis=1)
    conf = (1.0 / s).astype(jnp.float32)
    key = jax.lax.bitcast_convert_type(conf, jnp.int32)
    iot = jax.lax.broadcasted_iota(jnp.int32, (_ROWS, _C), 1)
    pred = jnp.min(jnp.where(x == m, iot, _C), axis=1)
    lab = lab_ref[...].reshape(_ROWS)
    acc = (pred == lab).astype(jnp.int32)
    b = pl.program_id(0)
    gidx = b * _ROWS + jax.lax.broadcasted_iota(jnp.int32, (_ROWS,), 0)
    pay = gidx * 2 + acc
    key_ref[...] = key.reshape(1, 1, _ROWS)
    pay_ref[...] = pay.reshape(1, 1, _ROWS)


def _sc_dense_body(logits_hbm, labels_hbm, key_hbm, pay_hbm, xbuf, lbuf, kbuf,
                   pbuf, dsem):
    c = lax.axis_index("c")
    s = lax.axis_index("s")
    wid = c * 16 + s
    base0 = _NT + wid * _RW
    lane = lax.iota(jnp.int32, 16)
    perms = [lane ^ j for j in (8, 4, 2, 1)]
    gd = lax.GatherDimensionNumbers(
        offset_dims=(), collapsed_slice_dims=(0,), start_index_map=(0,))

    def _splat(v, op):
        for pm in perms:
            v = op(v, lax.gather(v, pm[:, None], gd, (1,),
                                 mode=lax.GatherScatterMode.PROMISE_IN_BOUNDS))
        return v

    neginf = jnp.full((16,), -jnp.inf, jnp.float32)
    nchunks = _C // 16
    tail_off = _C - 16
    tail_lo = nchunks * 16 - tail_off

    def block(b, _):
        base = base0 + b * _SC_BLK
        pltpu.sync_copy(logits_hbm.at[pl.ds(base, _SC_BLK)], xbuf)
        pltpu.sync_copy(labels_hbm.at[pl.ds(base, _SC_BLK)], lbuf)
        lab = lbuf[...]

        def row(r, carry):
            res_key, res_pay = carry

            def p1(i, m):
                return jnp.maximum(m, xbuf[r, pl.ds(i * 16, 16)])

            m = lax.fori_loop(0, nchunks, p1, neginf, unroll=8)
            vt = xbuf[r, pl.ds(tail_off, 16)]
            m = jnp.maximum(m, jnp.where(lane >= tail_lo, vt, neginf))
            mb = _splat(m, jnp.maximum)

            def p2(i, carry2):
                ssum, pred = carry2
                v = xbuf[r, pl.ds(i * 16, 16)]
                ssum = ssum + jnp.exp(v - mb)
                pred = jnp.minimum(pred, jnp.where(v == mb, lane + i * 16, _C))
                return (ssum, pred)

            ssum, pred = lax.fori_loop(
                0, nchunks, p2,
                (jnp.zeros((16,), jnp.float32), jnp.full((16,), _C, jnp.int32)),
                unroll=8)
            et = jnp.exp(vt - mb)
            ssum = ssum + jnp.where(lane >= tail_lo, et, 0.0)
            pred = jnp.minimum(
                pred,
                jnp.where(vt == mb,
                          jnp.where(lane >= tail_lo, lane + tail_off, _C), _C))
            sum_s = _splat(ssum, jnp.add)
            pred_s = _splat(pred, jnp.minimum)
            accb = jnp.where(pred_s == lab, 1, 0)
            pay = (base + lane) * 2 + accb
            sel = lane == r
            res_key = jnp.where(sel, sum_s, res_key)
            res_pay = jnp.where(sel, pay, res_pay)
            return (res_key, res_pay)

        res_key, res_pay = lax.fori_loop(
            0, _SC_BLK, row,
            (jnp.zeros((16,), jnp.float32), jnp.zeros((16,), jnp.int32)))
        kbuf[pl.ds(b * _SC_BLK, _SC_BLK)] = res_key
        pbuf[pl.ds(b * _SC_BLK, _SC_BLK)] = res_pay
        return 0

    lax.fori_loop(0, _RW // _SC_BLK, block, 0)
    pltpu.sync_copy(kbuf, key_hbm.at[pl.ds(wid * _RW, _RW)])
    pltpu.sync_copy(pbuf, pay_hbm.at[pl.ds(wid * _RW, _RW)])


def _sort_body(key_ref, pay_ref, ece_ref, mce_ref):
    k = key_ref[...]
    p = pay_ref[...]
    row = jax.lax.broadcasted_iota(jnp.int32, (_SR, _SL), 0)
    col = jax.lax.broadcasted_iota(jnp.int32, (_SR, _SL), 1)
    flat = row * _SL + col
    for lk in range(1, 17):
        bigk = 1 << lk
        desc = (flat & bigk) != 0
        for lj in range(lk - 1, -1, -1):
            j = 1 << lj
            if j >= _SL:
                ax, d, sz = 0, j // _SL, _SR
            else:
                ax, d, sz = 1, j, _SL
            bit = (flat & j) != 0
            kf = pltpu.roll(k, sz - d, ax)
            kb = pltpu.roll(k, d, ax)
            pf = pltpu.roll(p, sz - d, ax)
            pb = pltpu.roll(p, d, ax)
            kp = jnp.where(bit, kb, kf)
            pp = jnp.where(bit, pb, pf)
            cp = (k > kp) | ((k == kp) & (p > pp))
            take = (cp ^ bit) ^ desc
            k = jnp.where(take, kp, k)
            p = jnp.where(take, pp, p)
    conf = jax.lax.bitcast_convert_type(k, jnp.float32)
    acc = (p & 1).astype(jnp.float32)
    ece = jnp.float32(0.0)
    mce = jnp.float32(0.0)
    for low in _BIN_LOWER:
        msk = (flat >= low) & (flat < low + _BIN_SIZE)
        c = jnp.sum(jnp.where(msk, conf, 0.0))
        a = jnp.sum(jnp.where(msk, acc, 0.0))
        ce = jnp.abs(c - a) / float(_BIN_SIZE)
        ece = ece + ce
        mce = jnp.maximum(mce, ce)
    ece_ref[...] = jnp.broadcast_to(ece / _NBINS, (1, 1))
    mce_ref[...] = jnp.broadcast_to(mce, (1, 1))


def kernel(logits, labels):
    lab3 = labels[:_NT].reshape(_G, 1, _ROWS)
    key_sc, pay_sc = pl.kernel(
        _sc_dense_body,
        out_type=[
            jax.ShapeDtypeStruct((_NSC,), jnp.float32),
            jax.ShapeDtypeStruct((_NSC,), jnp.int32),
        ],
        mesh=plsc.VectorSubcoreMesh(core_axis_name="c", subcore_axis_name="s"),
        cost_estimate=pl.CostEstimate(
            flops=200_000_000, transcendentals=16_000_000,
            bytes_accessed=70_000_000),
        scratch_types=[
            pltpu.VMEM((_SC_BLK, _C), jnp.float32),
            pltpu.VMEM((_SC_BLK,), jnp.int32),
            pltpu.VMEM((_RW,), jnp.float32),
            pltpu.VMEM((_RW,), jnp.int32),
            pltpu.SemaphoreType.DMA,
        ],
    )(logits, labels)
    key3, pay3 = pl.pallas_call(
        _dense_body,
        grid=(_G,),
        in_specs=[
            pl.BlockSpec((1, 1, _ROWS), lambda i: (i, 0, 0)),
            pl.BlockSpec((_ROWS, _C), lambda i: (i, 0)),
        ],
        out_specs=[
            pl.BlockSpec((1, 1, _ROWS), lambda i: (i, 0, 0)),
            pl.BlockSpec((1, 1, _ROWS), lambda i: (i, 0, 0)),
        ],
        out_shape=[
            jax.ShapeDtypeStruct((_G, 1, _ROWS), jnp.int32),
            jax.ShapeDtypeStruct((_G, 1, _ROWS), jnp.int32),
        ],
    )(lab3, logits)
    key_sc = lax.bitcast_convert_type(
        (1.0 / key_sc).astype(jnp.float32), jnp.int32)
    key_all = jnp.concatenate([key3.reshape(_NT), key_sc])
    pay_all = jnp.concatenate([pay3.reshape(_NT), pay_sc])
    if _DENSE_ONLY:
        return (jnp.sum(key_all).astype(jnp.float32), jnp.sum(pay_all).astype(jnp.float32))
    ece, mce = pl.pallas_call(
        _sort_body,
        out_shape=[
            jax.ShapeDtypeStruct((1, 1), jnp.float32),
            jax.ShapeDtypeStruct((1, 1), jnp.float32),
        ],
    )(key_all.reshape(_SR, _SL), pay_all.reshape(_SR, _SL))
    return (ece[0, 0], mce[0, 0])


# speedup vs baseline: 1.0021x; 1.0021x over previous
import numpy as np
import jax
import jax.numpy as jnp
from jax import lax
from jax.experimental import pallas as pl
from jax.experimental.pallas import tpu as pltpu
from jax.experimental.pallas import tpu_sc as plsc

_N = 65536
_C = 1000
_NBINS = 20
_BIN_SIZE = _N // _NBINS
_BIN_LOWER = tuple(
    int(v) for v in np.linspace(0, _N - _BIN_SIZE, _NBINS).astype(np.int32)
)

_DENSE_ONLY = False
_NT = 49152
_NSC = _N - _NT
_W = 32
_RW = _NSC // _W
_SC_BLK = 16
_ROWS = 4096
_G = _NT // _ROWS
_SR, _SL = 512, 128


def _dense_body(lab_ref, x_ref, key_ref, pay_ref):
    x = x_ref[...]
    m = jnp.max(x, axis=1, keepdims=True)
    s = jnp.sum(jnp.exp(x - m), axis=1)
    conf = (1.0 / s).astype(jnp.float32)
    key = jax.lax.bitcast_convert_type(conf, jnp.int32)
    iot = jax.lax.broadcasted_iota(jnp.int32, (_ROWS, _C), 1)
    pred = jnp.min(jnp.where(x == m, iot, _C), axis=1)
    lab = lab_ref[...].reshape(_ROWS)
    acc = (pred == lab).astype(jnp.int32)
    b = pl.program_id(0)
    gidx = b * _ROWS + jax.lax.broadcasted_iota(jnp.int32, (_ROWS,), 0)
    pay = gidx * 2 + acc
    key_ref[...] = key.reshape(1, 1, _ROWS)
    pay_ref[...] = pay.reshape(1, 1, _ROWS)


def _sc_dense_body(logits_hbm, labels_hbm, key_hbm, pay_hbm, xbuf, lbuf, kbuf,
                   pbuf, dsem):
    c = lax.axis_index("c")
    s = lax.axis_index("s")
    wid = c * 16 + s
    base0 = _NT + wid * _RW
    lane = lax.iota(jnp.int32, 16)
    perms = [lane ^ j for j in (8, 4, 2, 1)]
    gd = lax.GatherDimensionNumbers(
        offset_dims=(), collapsed_slice_dims=(0,), start_index_map=(0,))

    def _splat(v, op):
        for pm in perms:
            v = op(v, lax.gather(v, pm[:, None], gd, (1,),
                                 mode=lax.GatherScatterMode.PROMISE_IN_BOUNDS))
        return v

    neginf = jnp.full((16,), -jnp.inf, jnp.float32)
    nchunks = _C // 16
    tail_off = _C - 16
    tail_lo = nchunks * 16 - tail_off

    def block(b, _):
        base = base0 + b * _SC_BLK
        pltpu.sync_copy(logits_hbm.at[pl.ds(base, _SC_BLK)], xbuf)
        pltpu.sync_copy(labels_hbm.at[pl.ds(base, _SC_BLK)], lbuf)
        lab = lbuf[...]

        def row(r, carry):
            res_key, res_pay = carry

            def p1(i, m):
                return jnp.maximum(m, xbuf[r, pl.ds(i * 16, 16)])

            m = lax.fori_loop(0, nchunks, p1, neginf, unroll=8)
            vt = xbuf[r, pl.ds(tail_off, 16)]
            m = jnp.maximum(m, jnp.where(lane >= tail_lo, vt, neginf))
            mb = _splat(m, jnp.maximum)

            def p2(i, carry2):
                ssum, pred = carry2
                v = xbuf[r, pl.ds(i * 16, 16)]
                ssum = ssum + jnp.exp(v - mb)
                pred = jnp.minimum(pred, jnp.where(v == mb, lane + i * 16, _C))
                return (ssum, pred)

            ssum, pred = lax.fori_loop(
                0, nchunks, p2,
                (jnp.zeros((16,), jnp.float32), jnp.full((16,), _C, jnp.int32)),
                unroll=8)
            et = jnp.exp(vt - mb)
            ssum = ssum + jnp.where(lane >= tail_lo, et, 0.0)
            pred = jnp.minimum(
                pred,
                jnp.where(vt == mb,
                          jnp.where(lane >= tail_lo, lane + tail_off, _C), _C))
            sum_s = _splat(ssum, jnp.add)
            pred_s = _splat(pred, jnp.minimum)
            accb = jnp.where(pred_s == lab, 1, 0)
            pay = (base + lane) * 2 + accb
            sel = lane == r
            res_key = jnp.where(sel, sum_s, res_key)
            res_pay = jnp.where(sel, pay, res_pay)
            return (res_key, res_pay)

        res_key, res_pay = lax.fori_loop(
            0, _SC_BLK, row,
            (jnp.zeros((16,), jnp.float32), jnp.zeros((16,), jnp.int32)))
        kbuf[pl.ds(b * _SC_BLK, _SC_BLK)] = res_key
        pbuf[pl.ds(b * _SC_BLK, _SC_BLK)] = res_pay
        return 0

    lax.fori_loop(0, _RW // _SC_BLK, block, 0)
    pltpu.sync_copy(kbuf, key_hbm.at[pl.ds(wid * _RW, _RW)])
    pltpu.sync_copy(pbuf, pay_hbm.at[pl.ds(wid * _RW, _RW)])


def _sort_body(key_ref, pay_ref, ece_ref, mce_ref):
    k = key_ref[...]
    p = pay_ref[...]
    row = jax.lax.broadcasted_iota(jnp.int32, (_SR, _SL), 0)
    col = jax.lax.broadcasted_iota(jnp.int32, (_SR, _SL), 1)
    flat = row * _SL + col
    for lk in range(1, 17):
        bigk = 1 << lk
        desc = (flat & bigk) != 0
        for lj in range(lk - 1, -1, -1):
            j = 1 << lj
            if j >= _SL:
                ax, d, sz = 0, j // _SL, _SR
            else:
                ax, d, sz = 1, j, _SL
            bit = (flat & j) != 0
            kf = pltpu.roll(k, sz - d, ax)
            kb = pltpu.roll(k, d, ax)
            pf = pltpu.roll(p, sz - d, ax)
            pb = pltpu.roll(p, d, ax)
            kp = jnp.where(bit, kb, kf)
            pp = jnp.where(bit, pb, pf)
            cp = (k > kp) | ((k == kp) & (p > pp))
            take = (cp ^ bit) ^ desc
            k = jnp.where(take, kp, k)
            p = jnp.where(take, pp, p)
    conf = jax.lax.bitcast_convert_type(k, jnp.float32)
    acc = (p & 1).astype(jnp.float32)
    ece = jnp.float32(0.0)
    mce = jnp.float32(0.0)
    for low in _BIN_LOWER:
        msk = (flat >= low) & (flat < low + _BIN_SIZE)
        c = jnp.sum(jnp.where(msk, conf, 0.0))
        a = jnp.sum(jnp.where(msk, acc, 0.0))
        ce = jnp.abs(c - a) / float(_BIN_SIZE)
        ece = ece + ce
        mce = jnp.maximum(mce, ce)
    ece_ref[...] = jnp.broadcast_to(ece / _NBINS, (1, 1))
    mce_ref[...] = jnp.broadcast_to(mce, (1, 1))


def kernel(logits, labels):
    lab3 = labels[:_NT].reshape(_G, 1, _ROWS)
    key3, pay3 = pl.pallas_call(
        _dense_body,
        grid=(_G,),
        in_specs=[
            pl.BlockSpec((1, 1, _ROWS), lambda i: (i, 0, 0)),
            pl.BlockSpec((_ROWS, _C), lambda i: (i, 0)),
        ],
        out_specs=[
            pl.BlockSpec((1, 1, _ROWS), lambda i: (i, 0, 0)),
            pl.BlockSpec((1, 1, _ROWS), lambda i: (i, 0, 0)),
        ],
        out_shape=[
            jax.ShapeDtypeStruct((_G, 1, _ROWS), jnp.int32),
            jax.ShapeDtypeStruct((_G, 1, _ROWS), jnp.int32),
        ],
    )(lab3, logits)
    key_sc, pay_sc = pl.kernel(
        _sc_dense_body,
        out_type=[
            jax.ShapeDtypeStruct((_NSC,), jnp.float32),
            jax.ShapeDtypeStruct((_NSC,), jnp.int32),
        ],
        mesh=plsc.VectorSubcoreMesh(core_axis_name="c", subcore_axis_name="s"),
        cost_estimate=pl.CostEstimate(
            flops=200_000_000, transcendentals=16_000_000,
            bytes_accessed=70_000_000),
        scratch_types=[
            pltpu.VMEM((_SC_BLK, _C), jnp.float32),
            pltpu.VMEM((_SC_BLK,), jnp.int32),
            pltpu.VMEM((_RW,), jnp.float32),
            pltpu.VMEM((_RW,), jnp.int32),
            pltpu.SemaphoreType.DMA,
        ],
    )(logits, labels)
    key_sc = lax.bitcast_convert_type(
        (1.0 / key_sc).astype(jnp.float32), jnp.int32)
    key_all = jnp.concatenate([key3.reshape(_NT), key_sc])
    pay_all = jnp.concatenate([pay3.reshape(_NT), pay_sc])
    if _DENSE_ONLY:
        return (jnp.sum(key_all).astype(jnp.float32), jnp.sum(pay_all).astype(jnp.float32))
    ece, mce = pl.pallas_call(
        _sort_body,
        out_shape=[
            jax.ShapeDtypeStruct((1, 1), jnp.float32),
            jax.ShapeDtypeStruct((1, 1), jnp.float32),
        ],
    )(key_all.reshape(_SR, _SL), pay_all.reshape(_SR, _SL))
    return (ece[0, 0], mce[0, 0])


# speedup vs baseline: 1.1120x; 1.1097x over previous
import numpy as np
import jax
import jax.numpy as jnp
from jax.experimental import pallas as pl
from jax.experimental.pallas import tpu as pltpu

_N = 65536
_C = 1000
_NBINS = 20
_BIN_SIZE = _N // _NBINS
_BIN_LOWER = tuple(
    int(v) for v in np.linspace(0, _N - _BIN_SIZE, _NBINS).astype(np.int32)
)

_ROWS = 4096
_G = _N // _ROWS
_SR, _SL = 512, 128


def _dense_body(lab_ref, x_ref, key_ref, pay_ref):
    x = x_ref[...]
    m = jnp.max(x, axis=1, keepdims=True)
    s = jnp.sum(jnp.exp(x - m), axis=1)
    conf = (1.0 / s).astype(jnp.float32)
    key = jax.lax.bitcast_convert_type(conf, jnp.int32)
    iot = jax.lax.broadcasted_iota(jnp.int32, (_ROWS, _C), 1)
    pred = jnp.min(jnp.where(x == m, iot, _C), axis=1)
    lab = lab_ref[...].reshape(_ROWS)
    acc = (pred == lab).astype(jnp.int32)
    b = pl.program_id(0)
    gidx = b * _ROWS + jax.lax.broadcasted_iota(jnp.int32, (_ROWS,), 0)
    pay = gidx * 2 + acc
    key_ref[...] = key.reshape(1, 1, _ROWS)
    pay_ref[...] = pay.reshape(1, 1, _ROWS)


def _sort_body(key_ref, pay_ref, ece_ref, mce_ref):
    k = key_ref[...]
    p = pay_ref[...]
    row = jax.lax.broadcasted_iota(jnp.int32, (_SR, _SL), 0)
    col = jax.lax.broadcasted_iota(jnp.int32, (_SR, _SL), 1)
    flat = row * _SL + col
    for lk in range(1, 17):
        bigk = 1 << lk
        desc = (flat & bigk) != 0
        for lj in range(lk - 1, -1, -1):
            j = 1 << lj
            if j >= _SL:
                ax, d, sz = 0, j // _SL, _SR
            else:
                ax, d, sz = 1, j, _SL
            bit = (flat & j) != 0
            kf = pltpu.roll(k, sz - d, ax)
            kb = pltpu.roll(k, d, ax)
            pf = pltpu.roll(p, sz - d, ax)
            pb = pltpu.roll(p, d, ax)
            kp = jnp.where(bit, kb, kf)
            pp = jnp.where(bit, pb, pf)
            cp = (k > kp) | ((k == kp) & (p > pp))
            take = (cp ^ bit) ^ desc
            k = jnp.where(take, kp, k)
            p = jnp.where(take, pp, p)
    conf = jax.lax.bitcast_convert_type(k, jnp.float32)
    acc = (p & 1).astype(jnp.float32)
    ece = jnp.float32(0.0)
    mce = jnp.float32(0.0)
    for low in _BIN_LOWER:
        msk = (flat >= low) & (flat < low + _BIN_SIZE)
        c = jnp.sum(jnp.where(msk, conf, 0.0))
        a = jnp.sum(jnp.where(msk, acc, 0.0))
        ce = jnp.abs(c - a) / float(_BIN_SIZE)
        ece = ece + ce
        mce = jnp.maximum(mce, ce)
    ece_ref[...] = jnp.broadcast_to(ece / _NBINS, (1, 1))
    mce_ref[...] = jnp.broadcast_to(mce, (1, 1))


def kernel(logits, labels):
    lab3 = labels.reshape(_G, 1, _ROWS)
    key3, pay3 = pl.pallas_call(
        _dense_body,
        grid=(_G,),
        in_specs=[
            pl.BlockSpec((1, 1, _ROWS), lambda i: (i, 0, 0)),
            pl.BlockSpec((_ROWS, _C), lambda i: (i, 0)),
        ],
        out_specs=[
            pl.BlockSpec((1, 1, _ROWS), lambda i: (i, 0, 0)),
            pl.BlockSpec((1, 1, _ROWS), lambda i: (i, 0, 0)),
        ],
        out_shape=[
            jax.ShapeDtypeStruct((_G, 1, _ROWS), jnp.int32),
            jax.ShapeDtypeStruct((_G, 1, _ROWS), jnp.int32),
        ],
    )(lab3, logits)
    ece, mce = pl.pallas_call(
        _sort_body,
        out_shape=[
            jax.ShapeDtypeStruct((1, 1), jnp.float32),
            jax.ShapeDtypeStruct((1, 1), jnp.float32),
        ],
    )(key3.reshape(_SR, _SL), pay3.reshape(_SR, _SL))
    return (ece[0, 0], mce[0, 0])
